# Initial kernel scaffold; baseline (speedup 1.0000x reference)
import jax
import jax.numpy as jnp
from jax.experimental import pallas as pl


def kernel(x, edge_index_1, edge_index_2, W1l, W1r, b1, W2l, W2r, b2):
    raise NotImplementedError("write your pallas kernel here")


# speedup vs baseline: 6.0321x; 6.0321x over previous
import functools

import jax
import jax.numpy as jnp
from jax import lax
from jax.experimental import pallas as pl
from jax.experimental.pallas import tpu as pltpu
from jax.experimental.pallas import tpu_sc as plsc

_N1, _N2 = 5000, 1000
_E1, _E2 = 320000, 160000
_D, _H = 128, 128
_NC, _NS = 2, 16
_NW = _NC * _NS
_ACC = 1024
_JUNK = 1016


@functools.lru_cache(maxsize=None)
def _make_sc_agg(num_edges, clamp_dst):
    k = 80 if clamp_dst else 40
    per_tile = num_edges // _NW
    n_chunks = per_tile // k
    wpt = _ACC // _NS
    assert per_tile % k == 0

    mesh = plsc.VectorSubcoreMesh(core_axis_name="c", subcore_axis_name="s")
    out_t = jax.ShapeDtypeStruct((_NC, _ACC, _D), jnp.float32)

    @functools.partial(
        pl.kernel,
        mesh=mesh,
        out_type=(out_t, out_t),
        scratch_types=[
            pltpu.VMEM((n_chunks, k), jnp.int32),
            pltpu.VMEM((n_chunks, k), jnp.int32),
            pltpu.VMEM((k, _D), jnp.float32),
            pltpu.VMEM((k, _D), jnp.float32),
            pltpu.VMEM((wpt, _D), jnp.float32),
            pltpu.VMEM_SHARED((_ACC, _D), jnp.float32),
            pltpu.VMEM_SHARED((_ACC, _D), jnp.float32),
            pltpu.SemaphoreType.DMA,
        ],
    )
    def sc_agg(table_hbm, src_hbm, dst_hbm, out_feat, out_cnt,
               src_v, dst_v, rows_v, ones_v, zero_v, facc, cacc, sem):
        cid = lax.axis_index("c")
        sid = lax.axis_index("s")
        wid = cid * _NS + sid

        def fill_row(r, carry):
            for c in range(_D // 16):
                zero_v[r % wpt, pl.ds(c * 16, 16)] = jnp.zeros((16,), jnp.float32)
                ones_v[r % k, pl.ds(c * 16, 16)] = jnp.ones((16,), jnp.float32)
            return carry
        lax.fori_loop(0, max(wpt, k), fill_row, 0)
        pltpu.sync_copy(zero_v, facc.at[pl.ds(sid * wpt, wpt)])
        pltpu.sync_copy(zero_v, cacc.at[pl.ds(sid * wpt, wpt)])
        plsc.subcore_barrier()

        pltpu.sync_copy(src_hbm.at[wid], src_v)
        pltpu.sync_copy(dst_hbm.at[wid], dst_v)
        if clamp_dst:
            def clamp_row(j, carry):
                for t in range(k // 16):
                    v = dst_v[j, pl.ds(t * 16, 16)]
                    dst_v[j, pl.ds(t * 16, 16)] = jnp.where(
                        v < _N2, v, jnp.full((16,), _JUNK, jnp.int32))
                return carry
            lax.fori_loop(0, n_chunks, clamp_row, 0)

        def body(j, carry):
            pltpu.async_copy(table_hbm.at[src_v.at[j]], rows_v, sem).wait()
            pltpu.sync_copy(rows_v, facc.at[dst_v.at[j]], add=True)
            pltpu.sync_copy(ones_v, cacc.at[dst_v.at[j]], add=True)
            return carry
        lax.fori_loop(0, n_chunks, body, 0)

        plsc.subcore_barrier()
        pltpu.sync_copy(facc.at[pl.ds(sid * wpt, wpt)],
                        out_feat.at[cid, pl.ds(sid * wpt, wpt)])
        pltpu.sync_copy(cacc.at[pl.ds(sid * wpt, wpt)],
                        out_cnt.at[cid, pl.ds(sid * wpt, wpt)])

    return sc_agg


def _dot(a, b):
    return jnp.dot(a, b, preferred_element_type=jnp.float32,
                   precision=lax.Precision.HIGHEST)


def _tc_layer1(agg_ref, cnt_ref, x_ref, wl_ref, wr_ref, b_ref, out_ref):
    agg = agg_ref[0] + agg_ref[1]
    cnt = cnt_ref[0, :, :1] + cnt_ref[1, :, :1]
    mean = agg / jnp.maximum(cnt, 1.0)
    h = _dot(x_ref[...], wl_ref[...]) + _dot(mean, wr_ref[...]) + b_ref[...]
    out_ref[...] = jnp.maximum(h, 0.0)


def _tc_layer2(agg_ref, cnt_ref, h_ref, wl_ref, wr_ref, b_ref, out_ref):
    agg = agg_ref[0] + agg_ref[1]
    cnt = cnt_ref[0, :, :1] + cnt_ref[1, :, :1]
    mean = agg / jnp.maximum(cnt, 1.0)
    out_ref[...] = (_dot(h_ref[...], wl_ref[...])
                    + _dot(mean, wr_ref[...]) + b_ref[...])


def kernel(x, edge_index_1, edge_index_2, W1l, W1r, b1, W2l, W2r, b2):
    f32 = jnp.float32

    k1 = 80
    s1 = edge_index_1[0].reshape(_NW, _E1 // _NW // k1, k1)
    d1 = edge_index_1[1].reshape(_NW, _E1 // _NW // k1, k1)
    feat1, cnt1 = _make_sc_agg(_E1, True)(x[:_N1], s1, d1)

    h = pl.pallas_call(
        _tc_layer1,
        out_shape=jax.ShapeDtypeStruct((_ACC, _H), f32),
    )(feat1, cnt1, x[:_ACC], W1l, W1r, b1.reshape(1, _H))

    k2 = 40
    s2 = edge_index_2[0].reshape(_NW, _E2 // _NW // k2, k2)
    d2 = edge_index_2[1].reshape(_NW, _E2 // _NW // k2, k2)
    feat2, cnt2 = _make_sc_agg(_E2, False)(h, s2, d2)

    out = pl.pallas_call(
        _tc_layer2,
        out_shape=jax.ShapeDtypeStruct((_ACC, _H), f32),
    )(feat2, cnt2, h, W2l, W2r, b2.reshape(1, _H))
    return out[:_N2]


# speedup vs baseline: 7.3718x; 1.2221x over previous
import functools

import jax
import jax.numpy as jnp
from jax import lax
from jax.experimental import pallas as pl
from jax.experimental.pallas import tpu as pltpu
from jax.experimental.pallas import tpu_sc as plsc

_N1, _N2 = 5000, 1000
_E1, _E2 = 320000, 160000
_D, _H = 128, 128
_NC, _NS = 2, 16
_NW = _NC * _NS
_ACC = 1024
_JUNK = 1016


@functools.lru_cache(maxsize=None)
def _make_sc_agg(num_edges, clamp_dst):
    k = 80 if clamp_dst else 40
    per_tile = num_edges // _NW
    n_chunks = per_tile // k
    wpt = _ACC // _NS
    assert per_tile % k == 0

    mesh = plsc.VectorSubcoreMesh(core_axis_name="c", subcore_axis_name="s")
    out_t = jax.ShapeDtypeStruct((_NC, _ACC, _D), jnp.float32)

    @functools.partial(
        pl.kernel,
        mesh=mesh,
        out_type=(out_t, out_t),
        scratch_types=[
            pltpu.VMEM((n_chunks, k), jnp.int32),
            pltpu.VMEM((n_chunks, k), jnp.int32),
            pltpu.VMEM((k, _D), jnp.float32),
            pltpu.VMEM((k, _D), jnp.float32),
            pltpu.VMEM((k, _D), jnp.float32),
            pltpu.VMEM((wpt, _D), jnp.float32),
            pltpu.VMEM_SHARED((_ACC, _D), jnp.float32),
            pltpu.VMEM_SHARED((_ACC, _D), jnp.float32),
            pltpu.SemaphoreType.DMA,
            pltpu.SemaphoreType.DMA,
        ],
    )
    def sc_agg(table_hbm, src_hbm, dst_hbm, out_feat, out_cnt,
               src_v, dst_v, rows0_v, rows1_v, ones_v, zero_v,
               facc, cacc, sem0, sem1):
        cid = lax.axis_index("c")
        sid = lax.axis_index("s")
        wid = cid * _NS + sid

        def fill_row(r, carry):
            for c in range(_D // 16):
                zero_v[r % wpt, pl.ds(c * 16, 16)] = jnp.zeros((16,), jnp.float32)
                ones_v[r % k, pl.ds(c * 16, 16)] = jnp.ones((16,), jnp.float32)
            return carry
        lax.fori_loop(0, max(wpt, k), fill_row, 0)
        pltpu.sync_copy(zero_v, facc.at[pl.ds(sid * wpt, wpt)])
        pltpu.sync_copy(zero_v, cacc.at[pl.ds(sid * wpt, wpt)])
        plsc.subcore_barrier()

        pltpu.sync_copy(src_hbm.at[wid], src_v)
        pltpu.sync_copy(dst_hbm.at[wid], dst_v)
        if clamp_dst:
            def clamp_row(j, carry):
                for t in range(k // 16):
                    v = dst_v[j, pl.ds(t * 16, 16)]
                    dst_v[j, pl.ds(t * 16, 16)] = jnp.where(
                        v < _N2, v, jnp.full((16,), _JUNK, jnp.int32))
                return carry
            lax.fori_loop(0, n_chunks, clamp_row, 0)

        pltpu.async_copy(table_hbm.at[src_v.at[0]], rows0_v, sem0)

        def body(j, carry):
            nxt = j + 1
            @pl.when(jnp.logical_and(nxt < n_chunks, nxt % 2 == 0))
            def _():
                pltpu.async_copy(table_hbm.at[src_v.at[nxt]], rows0_v, sem0)
            @pl.when(jnp.logical_and(nxt < n_chunks, nxt % 2 == 1))
            def _():
                pltpu.async_copy(table_hbm.at[src_v.at[nxt]], rows1_v, sem1)
            pltpu.sync_copy(ones_v, cacc.at[dst_v.at[j]], add=True)
            @pl.when(j % 2 == 0)
            def _():
                pltpu.make_async_copy(table_hbm.at[src_v.at[j]],
                                      rows0_v, sem0).wait()
                pltpu.sync_copy(rows0_v, facc.at[dst_v.at[j]], add=True)
            @pl.when(j % 2 == 1)
            def _():
                pltpu.make_async_copy(table_hbm.at[src_v.at[j]],
                                      rows1_v, sem1).wait()
                pltpu.sync_copy(rows1_v, facc.at[dst_v.at[j]], add=True)
            return carry
        lax.fori_loop(0, n_chunks, body, 0)

        plsc.subcore_barrier()
        pltpu.sync_copy(facc.at[pl.ds(sid * wpt, wpt)],
                        out_feat.at[cid, pl.ds(sid * wpt, wpt)])
        pltpu.sync_copy(cacc.at[pl.ds(sid * wpt, wpt)],
                        out_cnt.at[cid, pl.ds(sid * wpt, wpt)])

    return sc_agg


def _dot(a, b):
    return jnp.dot(a, b, preferred_element_type=jnp.float32,
                   precision=lax.Precision.HIGHEST)


def _tc_layer1(agg_ref, cnt_ref, x_ref, wl_ref, wr_ref, b_ref, out_ref):
    agg = agg_ref[0] + agg_ref[1]
    cnt = cnt_ref[0, :, :1] + cnt_ref[1, :, :1]
    mean = agg / jnp.maximum(cnt, 1.0)
    h = _dot(x_ref[...], wl_ref[...]) + _dot(mean, wr_ref[...]) + b_ref[...]
    out_ref[...] = jnp.maximum(h, 0.0)


def _tc_layer2(agg_ref, cnt_ref, h_ref, wl_ref, wr_ref, b_ref, out_ref):
    agg = agg_ref[0] + agg_ref[1]
    cnt = cnt_ref[0, :, :1] + cnt_ref[1, :, :1]
    mean = agg / jnp.maximum(cnt, 1.0)
    out_ref[...] = (_dot(h_ref[...], wl_ref[...])
                    + _dot(mean, wr_ref[...]) + b_ref[...])


def kernel(x, edge_index_1, edge_index_2, W1l, W1r, b1, W2l, W2r, b2):
    f32 = jnp.float32

    k1 = 80
    s1 = edge_index_1[0].reshape(_NW, _E1 // _NW // k1, k1)
    d1 = edge_index_1[1].reshape(_NW, _E1 // _NW // k1, k1)
    feat1, cnt1 = _make_sc_agg(_E1, True)(x[:_N1], s1, d1)

    h = pl.pallas_call(
        _tc_layer1,
        out_shape=jax.ShapeDtypeStruct((_ACC, _H), f32),
    )(feat1, cnt1, x[:_ACC], W1l, W1r, b1.reshape(1, _H))

    k2 = 40
    s2 = edge_index_2[0].reshape(_NW, _E2 // _NW // k2, k2)
    d2 = edge_index_2[1].reshape(_NW, _E2 // _NW // k2, k2)
    feat2, cnt2 = _make_sc_agg(_E2, False)(h, s2, d2)

    out = pl.pallas_call(
        _tc_layer2,
        out_shape=jax.ShapeDtypeStruct((_ACC, _H), f32),
    )(feat2, cnt2, h, W2l, W2r, b2.reshape(1, _H))
    return out[:_N2]


# speedup vs baseline: 7.3895x; 1.0024x over previous
import functools

import jax
import jax.numpy as jnp
from jax import lax
from jax.experimental import pallas as pl
from jax.experimental.pallas import tpu as pltpu
from jax.experimental.pallas import tpu_sc as plsc

_N1, _N2 = 5000, 1000
_E1, _E2 = 320000, 160000
_D, _H = 128, 128
_NC, _NS = 2, 16
_NW = _NC * _NS
_ACC = 1024
_JUNK = 1016


@functools.lru_cache(maxsize=None)
def _make_sc_agg(num_edges, clamp_dst):
    k = 80 if clamp_dst else 40
    per_tile = num_edges // _NW
    n_chunks = per_tile // k
    wpt = _ACC // _NS
    assert per_tile % k == 0

    mesh = plsc.VectorSubcoreMesh(core_axis_name="c", subcore_axis_name="s")
    out_t = jax.ShapeDtypeStruct((_NC, _ACC, _D), jnp.float32)

    @functools.partial(
        pl.kernel,
        mesh=mesh,
        out_type=(out_t, out_t),
        scratch_types=[
            pltpu.VMEM((n_chunks, k), jnp.int32),
            pltpu.VMEM((n_chunks, k), jnp.int32),
            pltpu.VMEM((k, _D), jnp.float32),
            pltpu.VMEM((k, _D), jnp.float32),
            pltpu.VMEM((k, _D), jnp.float32),
            pltpu.VMEM((wpt, _D), jnp.float32),
            pltpu.VMEM_SHARED((_ACC, _D), jnp.float32),
            pltpu.VMEM_SHARED((_ACC, _D), jnp.float32),
            pltpu.SemaphoreType.DMA,
            pltpu.SemaphoreType.DMA,
        ],
    )
    def sc_agg(table_hbm, src_hbm, dst_hbm, out_feat, out_cnt,
               src_v, dst_v, rows0_v, rows1_v, ones_v, zero_v,
               facc, cacc, sem0, sem1):
        cid = lax.axis_index("c")
        sid = lax.axis_index("s")
        wid = cid * _NS + sid

        def fill_row(r, carry):
            for c in range(_D // 16):
                zero_v[r % wpt, pl.ds(c * 16, 16)] = jnp.zeros((16,), jnp.float32)
                ones_v[r % k, pl.ds(c * 16, 16)] = jnp.ones((16,), jnp.float32)
            return carry
        lax.fori_loop(0, max(wpt, k), fill_row, 0)
        pltpu.sync_copy(zero_v, facc.at[pl.ds(sid * wpt, wpt)])
        pltpu.sync_copy(zero_v, cacc.at[pl.ds(sid * wpt, wpt)])
        plsc.subcore_barrier()

        pltpu.sync_copy(src_hbm.at[wid], src_v)
        pltpu.sync_copy(dst_hbm.at[wid], dst_v)
        if clamp_dst:
            junk16 = jnp.full((16,), _JUNK, jnp.int32)

            def clamp_row(j, carry):
                for t in range(k // 16):
                    d16 = dst_v[j, pl.ds(t * 16, 16)]
                    dst_v[j, pl.ds(t * 16, 16)] = jnp.where(d16 < _N2, d16,
                                                            junk16)
                return carry
            lax.fori_loop(0, n_chunks, clamp_row, 0)

        pltpu.async_copy(table_hbm.at[src_v.at[0]], rows0_v, sem0)

        def body(j, carry):
            nxt = j + 1
            @pl.when(jnp.logical_and(nxt < n_chunks, nxt % 2 == 0))
            def _():
                pltpu.async_copy(table_hbm.at[src_v.at[nxt]], rows0_v, sem0)
            @pl.when(jnp.logical_and(nxt < n_chunks, nxt % 2 == 1))
            def _():
                pltpu.async_copy(table_hbm.at[src_v.at[nxt]], rows1_v, sem1)
            pltpu.sync_copy(ones_v, cacc.at[dst_v.at[j]], add=True)
            @pl.when(j % 2 == 0)
            def _():
                pltpu.make_async_copy(table_hbm.at[src_v.at[j]],
                                      rows0_v, sem0).wait()
                pltpu.sync_copy(rows0_v, facc.at[dst_v.at[j]], add=True)
            @pl.when(j % 2 == 1)
            def _():
                pltpu.make_async_copy(table_hbm.at[src_v.at[j]],
                                      rows1_v, sem1).wait()
                pltpu.sync_copy(rows1_v, facc.at[dst_v.at[j]], add=True)
            return carry
        lax.fori_loop(0, n_chunks, body, 0)

        plsc.subcore_barrier()
        pltpu.sync_copy(facc.at[pl.ds(sid * wpt, wpt)],
                        out_feat.at[cid, pl.ds(sid * wpt, wpt)])
        pltpu.sync_copy(cacc.at[pl.ds(sid * wpt, wpt)],
                        out_cnt.at[cid, pl.ds(sid * wpt, wpt)])

    return sc_agg


def _dot(a, b):
    return jnp.dot(a, b, preferred_element_type=jnp.float32,
                   precision=lax.Precision.HIGHEST)


def _tc_layer1(agg_ref, cnt_ref, x_ref, wl_ref, wr_ref, b_ref, out_ref):
    agg = agg_ref[0] + agg_ref[1]
    cnt = cnt_ref[0, :, :1] + cnt_ref[1, :, :1]
    mean = agg / jnp.maximum(cnt, 1.0)
    h = _dot(x_ref[...], wl_ref[...]) + _dot(mean, wr_ref[...]) + b_ref[...]
    out_ref[...] = jnp.maximum(h, 0.0)


def _tc_layer2(agg_ref, cnt_ref, h_ref, wl_ref, wr_ref, b_ref, out_ref):
    agg = agg_ref[0] + agg_ref[1]
    cnt = cnt_ref[0, :, :1] + cnt_ref[1, :, :1]
    mean = agg / jnp.maximum(cnt, 1.0)
    out_ref[...] = (_dot(h_ref[...], wl_ref[...])
                    + _dot(mean, wr_ref[...]) + b_ref[...])


def kernel(x, edge_index_1, edge_index_2, W1l, W1r, b1, W2l, W2r, b2):
    f32 = jnp.float32

    k1 = 80
    s1 = edge_index_1[0].reshape(_NW, _E1 // _NW // k1, k1)
    d1 = edge_index_1[1].reshape(_NW, _E1 // _NW // k1, k1)
    feat1, cnt1 = _make_sc_agg(_E1, True)(x[:_N1], s1, d1)

    h = pl.pallas_call(
        _tc_layer1,
        out_shape=jax.ShapeDtypeStruct((_ACC, _H), f32),
    )(feat1, cnt1, x[:_ACC], W1l, W1r, b1.reshape(1, _H))

    k2 = 40
    s2 = edge_index_2[0].reshape(_NW, _E2 // _NW // k2, k2)
    d2 = edge_index_2[1].reshape(_NW, _E2 // _NW // k2, k2)
    feat2, cnt2 = _make_sc_agg(_E2, False)(h, s2, d2)

    out = pl.pallas_call(
        _tc_layer2,
        out_shape=jax.ShapeDtypeStruct((_ACC, _H), f32),
    )(feat2, cnt2, h, W2l, W2r, b2.reshape(1, _H))
    return out[:_N2]


# speedup vs baseline: 7.5466x; 1.0213x over previous
import functools

import jax
import jax.numpy as jnp
from jax import lax
from jax.experimental import pallas as pl
from jax.experimental.pallas import tpu as pltpu
from jax.experimental.pallas import tpu_sc as plsc

_N1, _N2 = 5000, 1000
_E1, _E2 = 320000, 160000
_D, _H = 128, 128
_NC, _NS = 2, 16
_NW = _NC * _NS
_ACC = 1024
_JUNK = 1016


@functools.lru_cache(maxsize=None)
def _make_sc_agg(num_edges, clamp_dst, k):
    per_tile = num_edges // _NW
    n_chunks = per_tile // k
    wpt = _ACC // _NS
    assert per_tile % k == 0

    mesh = plsc.VectorSubcoreMesh(core_axis_name="c", subcore_axis_name="s")
    out_t = jax.ShapeDtypeStruct((_NC, _ACC, _D), jnp.float32)

    @functools.partial(
        pl.kernel,
        mesh=mesh,
        out_type=(out_t, out_t),
        scratch_types=[
            pltpu.VMEM((n_chunks, k), jnp.int32),
            pltpu.VMEM((n_chunks, k), jnp.int32),
            pltpu.VMEM((k, _D), jnp.float32),
            pltpu.VMEM((k, _D), jnp.float32),
            pltpu.VMEM((k, _D), jnp.float32),
            pltpu.VMEM((wpt, _D), jnp.float32),
            pltpu.VMEM_SHARED((_ACC, _D), jnp.float32),
            pltpu.VMEM_SHARED((_ACC, _D), jnp.float32),
            pltpu.SemaphoreType.DMA,
            pltpu.SemaphoreType.DMA,
        ],
    )
    def sc_agg(table_hbm, src_hbm, dst_hbm, out_feat, out_cnt,
               src_v, dst_v, rows0_v, rows1_v, ones_v, zero_v,
               facc, cacc, sem0, sem1):
        cid = lax.axis_index("c")
        sid = lax.axis_index("s")
        wid = cid * _NS + sid

        def fill_row(r, carry):
            for c in range(_D // 16):
                zero_v[r % wpt, pl.ds(c * 16, 16)] = jnp.zeros((16,), jnp.float32)
                ones_v[r % k, pl.ds(c * 16, 16)] = jnp.ones((16,), jnp.float32)
            return carry
        lax.fori_loop(0, max(wpt, k), fill_row, 0)
        pltpu.sync_copy(zero_v, facc.at[pl.ds(sid * wpt, wpt)])
        pltpu.sync_copy(zero_v, cacc.at[pl.ds(sid * wpt, wpt)])
        plsc.subcore_barrier()

        pltpu.sync_copy(src_hbm.at[wid], src_v)
        pltpu.sync_copy(dst_hbm.at[wid], dst_v)
        if clamp_dst:
            junk16 = jnp.full((16,), _JUNK, jnp.int32)

            def clamp_row(j, carry):
                for t in range(k // 16):
                    d16 = dst_v[j, pl.ds(t * 16, 16)]
                    dst_v[j, pl.ds(t * 16, 16)] = jnp.where(d16 < _N2, d16,
                                                            junk16)
                return carry
            lax.fori_loop(0, n_chunks, clamp_row, 0)

        pltpu.async_copy(table_hbm.at[src_v.at[0]], rows0_v, sem0)

        def body(j, carry):
            nxt = j + 1
            @pl.when(jnp.logical_and(nxt < n_chunks, nxt % 2 == 0))
            def _():
                pltpu.async_copy(table_hbm.at[src_v.at[nxt]], rows0_v, sem0)
            @pl.when(jnp.logical_and(nxt < n_chunks, nxt % 2 == 1))
            def _():
                pltpu.async_copy(table_hbm.at[src_v.at[nxt]], rows1_v, sem1)
            pltpu.sync_copy(ones_v, cacc.at[dst_v.at[j]], add=True)
            @pl.when(j % 2 == 0)
            def _():
                pltpu.make_async_copy(table_hbm.at[src_v.at[j]],
                                      rows0_v, sem0).wait()
                pltpu.sync_copy(rows0_v, facc.at[dst_v.at[j]], add=True)
            @pl.when(j % 2 == 1)
            def _():
                pltpu.make_async_copy(table_hbm.at[src_v.at[j]],
                                      rows1_v, sem1).wait()
                pltpu.sync_copy(rows1_v, facc.at[dst_v.at[j]], add=True)
            return carry
        lax.fori_loop(0, n_chunks, body, 0)

        plsc.subcore_barrier()
        pltpu.sync_copy(facc.at[pl.ds(sid * wpt, wpt)],
                        out_feat.at[cid, pl.ds(sid * wpt, wpt)])
        pltpu.sync_copy(cacc.at[pl.ds(sid * wpt, wpt)],
                        out_cnt.at[cid, pl.ds(sid * wpt, wpt)])

    return sc_agg


def _dot(a, b):
    return jnp.dot(a, b, preferred_element_type=jnp.float32,
                   precision=lax.Precision.HIGHEST)


def _tc_layer1(agg_ref, cnt_ref, x_ref, wl_ref, wr_ref, b_ref, out_ref):
    agg = agg_ref[0] + agg_ref[1]
    cnt = cnt_ref[0, :, :1] + cnt_ref[1, :, :1]
    mean = agg / jnp.maximum(cnt, 1.0)
    h = _dot(x_ref[...], wl_ref[...]) + _dot(mean, wr_ref[...]) + b_ref[...]
    out_ref[...] = jnp.maximum(h, 0.0)


def _tc_layer2(agg_ref, cnt_ref, h_ref, wl_ref, wr_ref, b_ref, out_ref):
    agg = agg_ref[0] + agg_ref[1]
    cnt = cnt_ref[0, :, :1] + cnt_ref[1, :, :1]
    mean = agg / jnp.maximum(cnt, 1.0)
    out_ref[...] = (_dot(h_ref[...], wl_ref[...])
                    + _dot(mean, wr_ref[...]) + b_ref[...])


def kernel(x, edge_index_1, edge_index_2, W1l, W1r, b1, W2l, W2r, b2):
    f32 = jnp.float32

    k1 = 100
    s1 = edge_index_1[0].reshape(_NW, _E1 // _NW // k1, k1)
    d1 = edge_index_1[1].reshape(_NW, _E1 // _NW // k1, k1)
    feat1, cnt1 = _make_sc_agg(_E1, True, k1)(x[:_N1], s1, d1)

    h = pl.pallas_call(
        _tc_layer1,
        out_shape=jax.ShapeDtypeStruct((_ACC, _H), f32),
    )(feat1, cnt1, x[:_ACC], W1l, W1r, b1.reshape(1, _H))

    k2 = 50
    s2 = edge_index_2[0].reshape(_NW, _E2 // _NW // k2, k2)
    d2 = edge_index_2[1].reshape(_NW, _E2 // _NW // k2, k2)
    feat2, cnt2 = _make_sc_agg(_E2, False, k2)(h, s2, d2)

    out = pl.pallas_call(
        _tc_layer2,
        out_shape=jax.ShapeDtypeStruct((_ACC, _H), f32),
    )(feat2, cnt2, h, W2l, W2r, b2.reshape(1, _H))
    return out[:_N2]


# speedup vs baseline: 7.6685x; 1.0161x over previous
import functools

import jax
import jax.numpy as jnp
from jax import lax
from jax.experimental import pallas as pl
from jax.experimental.pallas import tpu as pltpu
from jax.experimental.pallas import tpu_sc as plsc

_N1, _N2 = 5000, 1000
_E1, _E2 = 320000, 160000
_D, _H = 128, 128
_NC, _NS = 2, 16
_NW = _NC * _NS
_ACC = 1024
_JUNK = 1016


@functools.lru_cache(maxsize=None)
def _make_sc_agg(num_edges, clamp_dst, k):
    per_tile = num_edges // _NW
    n_chunks = per_tile // k
    wpt = _ACC // _NS
    assert per_tile % k == 0

    mesh = plsc.VectorSubcoreMesh(core_axis_name="c", subcore_axis_name="s")
    out_t = jax.ShapeDtypeStruct((_NC, _ACC, _D), jnp.float32)

    @functools.partial(
        pl.kernel,
        mesh=mesh,
        out_type=(out_t, out_t),
        scratch_types=[
            pltpu.VMEM((n_chunks, k), jnp.int32),
            pltpu.VMEM((n_chunks, k), jnp.int32),
            pltpu.VMEM((k, _D), jnp.float32),
            pltpu.VMEM((k, _D), jnp.float32),
            pltpu.VMEM((k, _D), jnp.float32),
            pltpu.VMEM((wpt, _D), jnp.float32),
            pltpu.VMEM_SHARED((_ACC, _D), jnp.float32),
            pltpu.VMEM_SHARED((_ACC, _D), jnp.float32),
            pltpu.SemaphoreType.DMA,
            pltpu.SemaphoreType.DMA,
        ],
    )
    def sc_agg(table_hbm, src_hbm, dst_hbm, out_feat, out_cnt,
               src_v, dst_v, rows0_v, rows1_v, ones_v, zero_v,
               facc, cacc, sem0, sem1):
        cid = lax.axis_index("c")
        sid = lax.axis_index("s")
        wid = cid * _NS + sid

        def fill_row(r, carry):
            for c in range(_D // 16):
                zero_v[r % wpt, pl.ds(c * 16, 16)] = jnp.zeros((16,), jnp.float32)
                ones_v[r % k, pl.ds(c * 16, 16)] = jnp.ones((16,), jnp.float32)
            return carry
        lax.fori_loop(0, max(wpt, k), fill_row, 0)
        pltpu.sync_copy(zero_v, facc.at[pl.ds(sid * wpt, wpt)])
        pltpu.sync_copy(zero_v, cacc.at[pl.ds(sid * wpt, wpt)])
        plsc.subcore_barrier()

        pltpu.sync_copy(src_hbm.at[wid], src_v)
        pltpu.sync_copy(dst_hbm.at[wid], dst_v)
        if clamp_dst:
            junk16 = jnp.full((16,), _JUNK, jnp.int32)

            def clamp_row(j, carry):
                for t in range(k // 16):
                    d16 = dst_v[j, pl.ds(t * 16, 16)]
                    dst_v[j, pl.ds(t * 16, 16)] = jnp.where(d16 < _N2, d16,
                                                            junk16)
                return carry
            lax.fori_loop(0, n_chunks, clamp_row, 0)

        pltpu.async_copy(table_hbm.at[src_v.at[0]], rows0_v, sem0)

        def body(j, carry):
            nxt = j + 1
            @pl.when(jnp.logical_and(nxt < n_chunks, nxt % 2 == 0))
            def _():
                pltpu.async_copy(table_hbm.at[src_v.at[nxt]], rows0_v, sem0)
            @pl.when(jnp.logical_and(nxt < n_chunks, nxt % 2 == 1))
            def _():
                pltpu.async_copy(table_hbm.at[src_v.at[nxt]], rows1_v, sem1)
            pltpu.sync_copy(ones_v, cacc.at[dst_v.at[j]], add=True)
            @pl.when(j % 2 == 0)
            def _():
                pltpu.make_async_copy(table_hbm.at[src_v.at[j]],
                                      rows0_v, sem0).wait()
                pltpu.sync_copy(rows0_v, facc.at[dst_v.at[j]], add=True)
            @pl.when(j % 2 == 1)
            def _():
                pltpu.make_async_copy(table_hbm.at[src_v.at[j]],
                                      rows1_v, sem1).wait()
                pltpu.sync_copy(rows1_v, facc.at[dst_v.at[j]], add=True)
            return carry
        lax.fori_loop(0, n_chunks, body, 0)

        plsc.subcore_barrier()
        pltpu.sync_copy(facc.at[pl.ds(sid * wpt, wpt)],
                        out_feat.at[cid, pl.ds(sid * wpt, wpt)])
        pltpu.sync_copy(cacc.at[pl.ds(sid * wpt, wpt)],
                        out_cnt.at[cid, pl.ds(sid * wpt, wpt)])

    return sc_agg


def _dot(a, b):
    return jnp.dot(a, b, preferred_element_type=jnp.float32,
                   precision=lax.Precision.HIGHEST)


def _tc_layer1(agg_ref, cnt_ref, x_ref, wl_ref, wr_ref, b_ref, out_ref):
    agg = agg_ref[0] + agg_ref[1]
    cnt = cnt_ref[0, :, :1] + cnt_ref[1, :, :1]
    mean = agg / jnp.maximum(cnt, 1.0)
    h = _dot(x_ref[...], wl_ref[...]) + _dot(mean, wr_ref[...]) + b_ref[...]
    out_ref[...] = jnp.maximum(h, 0.0)


def _tc_layer2(agg_ref, cnt_ref, h_ref, wl_ref, wr_ref, b_ref, out_ref):
    agg = agg_ref[0] + agg_ref[1]
    cnt = cnt_ref[0, :, :1] + cnt_ref[1, :, :1]
    mean = agg / jnp.maximum(cnt, 1.0)
    out_ref[...] = (_dot(h_ref[...], wl_ref[...])
                    + _dot(mean, wr_ref[...]) + b_ref[...])


def kernel(x, edge_index_1, edge_index_2, W1l, W1r, b1, W2l, W2r, b2):
    f32 = jnp.float32

    k1 = 125
    s1 = edge_index_1[0].reshape(_NW, _E1 // _NW // k1, k1)
    d1 = edge_index_1[1].reshape(_NW, _E1 // _NW // k1, k1)
    feat1, cnt1 = _make_sc_agg(_E1, True, k1)(x[:_N1], s1, d1)

    h = pl.pallas_call(
        _tc_layer1,
        out_shape=jax.ShapeDtypeStruct((_ACC, _H), f32),
    )(feat1, cnt1, x[:_ACC], W1l, W1r, b1.reshape(1, _H))

    k2 = 125
    s2 = edge_index_2[0].reshape(_NW, _E2 // _NW // k2, k2)
    d2 = edge_index_2[1].reshape(_NW, _E2 // _NW // k2, k2)
    feat2, cnt2 = _make_sc_agg(_E2, False, k2)(h, s2, d2)

    out = pl.pallas_call(
        _tc_layer2,
        out_shape=jax.ShapeDtypeStruct((_ACC, _H), f32),
    )(feat2, cnt2, h, W2l, W2r, b2.reshape(1, _H))
    return out[:_N2]
